# Initial kernel scaffold; baseline (speedup 1.0000x reference)
import jax
import jax.numpy as jnp
from jax.experimental import pallas as pl


def kernel(descriptors, elems, W, b):
    raise NotImplementedError("write your pallas kernel here")


# speedup vs baseline: 8.4958x; 8.4958x over previous
import functools

import jax
import jax.numpy as jnp
from jax import lax
from jax.experimental import pallas as pl
from jax.experimental.pallas import tpu as pltpu
from jax.experimental.pallas import tpu_sc as plsc

N = 100000
D = 128
N_TYPES = 4
L = 16
NC, NS = 2, 16
NW = NC * NS
CHUNK = 128
NFULL = N // CHUNK
TAIL = N - NFULL * CHUNK
ITERS = (NFULL + NW - 1) // NW
TAIL_WID = NFULL % NW


def _permute(x, idx):
    dnums = lax.GatherDimensionNumbers(
        offset_dims=(), collapsed_slice_dims=(0,), start_index_map=(0,))
    return lax.gather(x, idx[:, None], dnums, slice_sizes=(1,),
                      mode=lax.GatherScatterMode.PROMISE_IN_BOUNDS)


def _body(desc_hbm, elems_hbm, wt_hbm, out_hbm, dbuf, ebuf, wbuf, obuf):
    wid = lax.axis_index("s") * NC + lax.axis_index("c")
    pltpu.sync_copy(wt_hbm, wbuf)
    lane = lax.broadcasted_iota(jnp.int32, (L,), 0)

    def do_chunk(base, rows):
        pltpu.sync_copy(desc_hbm.at[pl.ds(base, rows), :],
                        dbuf.at[pl.ds(0, rows), :])
        pltpu.sync_copy(elems_hbm.at[pl.ds(base, rows)],
                        ebuf.at[pl.ds(0, rows)])

        def group_body(g, _):
            ev = ebuf[pl.ds(g * L, L)]
            res = jnp.zeros((L,), jnp.float32)
            for k in range(L):
                e = ev[k]
                r = g * L + k
                acc = dbuf[r, pl.ds(0, L)] * wbuf[e, pl.ds(0, L)]
                for j in range(1, D // L):
                    acc = acc + dbuf[r, pl.ds(j * L, L)] * wbuf[e, pl.ds(j * L, L)]
                for sh in (8, 4, 2, 1):
                    acc = acc + _permute(acc, lane ^ sh)
                res = jnp.where(lane == k, acc, res)
            obuf[pl.ds(g * L, L)] = res
            return 0

        lax.fori_loop(0, rows // L, group_body, 0)
        pltpu.sync_copy(obuf.at[pl.ds(0, rows)],
                        out_hbm.at[pl.ds(base, rows)])

    def iter_body(i, _):
        c = i * NW + wid

        @pl.when(c < NFULL)
        def _():
            do_chunk(c * CHUNK, CHUNK)

        return 0

    lax.fori_loop(0, ITERS, iter_body, 0)

    @pl.when(wid == TAIL_WID)
    def _():
        do_chunk(NFULL * CHUNK, TAIL)


_mesh = plsc.VectorSubcoreMesh(core_axis_name="c", subcore_axis_name="s")

_sc_dot = functools.partial(
    pl.kernel,
    mesh=_mesh,
    out_type=jax.ShapeDtypeStruct((N,), jnp.float32),
    scratch_types=[
        pltpu.VMEM((CHUNK, D), jnp.float32),
        pltpu.VMEM((CHUNK,), jnp.int32),
        pltpu.VMEM((N_TYPES, D), jnp.float32),
        pltpu.VMEM((CHUNK,), jnp.float32),
    ],
)(_body)


def kernel(descriptors, elems, W, b):
    wt = W.reshape(N_TYPES, D)
    dots = _sc_dot(descriptors, elems.astype(jnp.int32), wt)
    return dots.reshape(N, 1) + b
